# baseline (device time: 121438 ns/iter reference)
import contextlib

import jax
import jax.numpy as jnp
from jax import lax
from jax.experimental import pallas as pl
from jax.experimental.pallas import tpu as pltpu

_PROFILE_SCOPES = False


def _scope(name):
    if _PROFILE_SCOPES:
        return jax.named_scope(name)
    return contextlib.nullcontext()

N_DEV = 4
SQ = 2048
DM = 1024
H = 8
DH = 128
BLK = 64
R = 4
JB = 8
CHUNK = SQ // N_DEV
HALF = DM // 2
SCALE = 0.08838834764831843
MESH = pl.DeviceIdType.MESH


def _body(x_hbm, wq_ref, k_hbm, v_hbm, wo_ref, out_ref,
          xr, qr, kr, vr, o_buf, stats_r, stats_comm, obf,
          dma_sems, st_send, st_recv, send_cw, send_ccw,
          rs_cw, rs_ccw, ag_cw, ag_ccw,
          rs_recv_cw, rs_recv_ccw, ag_recv_cw, ag_recv_ccw):
    my = lax.axis_index("i")
    right = jnp.mod(my + 1, N_DEV)
    left = jnp.mod(my - 1, N_DEV)
    diag = jnp.mod(my + 2, N_DEV)

    def issue_copies(r):
        slot = r % 2
        copies = []
        for j in range(JB):
            b = r + R * j
            copies.append(pltpu.make_async_copy(
                x_hbm.at[0, pl.ds(b * BLK, BLK), :],
                xr.at[slot, pl.ds(j * BLK, BLK), :],
                dma_sems.at[slot, 0]))
            copies.append(pltpu.make_async_copy(
                k_hbm.at[0, pl.ds(b * BLK, BLK), :, :],
                kr.at[slot, pl.ds(j * BLK, BLK), :, :],
                dma_sems.at[slot, 1]))
            copies.append(pltpu.make_async_copy(
                v_hbm.at[0, pl.ds(b * BLK, BLK), :, :],
                vr.at[slot, pl.ds(j * BLK, BLK), :, :],
                dma_sems.at[slot, 2]))
        for c in copies:
            c.start()
        return copies

    pending = issue_copies(0)
    for r in range(R):
      with _scope(f"phA#r={r}"):
        slot = r % 2
        blocks = [r + R * j for j in range(JB)]
        nxt = issue_copies(r + 1) if r + 1 < R else []
        for c in pending:
            c.wait()
        pending = nxt

        qr[:, :] = jnp.dot(xr[slot], wq_ref[:, :],
                           preferred_element_type=jnp.float32)
        for h in range(H):
            cs = slice(h * DH, (h + 1) * DH)
            s = lax.dot_general(qr[:, cs], kr[slot, :, h, :],
                                (((1,), (1,)), ((), ())),
                                preferred_element_type=jnp.float32) * SCALE
            m = jnp.max(s, axis=1, keepdims=True)
            w = jnp.exp(s - m)
            l = jnp.sum(w, axis=1, keepdims=True)
            o = jnp.dot(w, vr[slot, :, h, :],
                        preferred_element_type=jnp.float32)
            for j, b in enumerate(blocks):
                o_buf[pl.ds(b * BLK, BLK), cs] = o[j * BLK:(j + 1) * BLK, :]
            stats_r[:, h:h + 1] = m
            stats_r[:, H + h:H + h + 1] = l
        for j, b in enumerate(blocks):
            stats_comm[0, pl.ds(b * BLK, BLK), :] = stats_r[pl.ds(j * BLK, BLK), :]

    with _scope("barrier"):
        barrier = pltpu.get_barrier_semaphore()
        for nbr in (left, right, diag):
            pl.semaphore_signal(barrier, inc=1, device_id=(nbr,),
                                device_id_type=MESH)
        pl.semaphore_wait(barrier, 3)

    with _scope("stats_x"):
        stat_sends = []
        for d in range(1, N_DEV):
            slot = N_DEV - d
            rdma = pltpu.make_async_remote_copy(
                src_ref=stats_comm.at[0],
                dst_ref=stats_comm.at[slot],
                send_sem=st_send.at[d - 1],
                recv_sem=st_recv.at[slot - 1],
                device_id=(jnp.mod(my + d, N_DEV),),
                device_id_type=MESH,
            )
            rdma.start()
            stat_sends.append(rdma)
        for k in range(1, N_DEV):
            pltpu.make_async_remote_copy(
                src_ref=stats_comm.at[k], dst_ref=stats_comm.at[k],
                send_sem=st_send.at[k - 1], recv_sem=st_recv.at[k - 1],
                device_id=(left,), device_id_type=MESH,
            ).wait_recv()
        for rdma in stat_sends:
            rdma.wait_send()

    with _scope("combine"):
        m_all = [stats_comm[s, :, 0:H] for s in range(N_DEV)]
        l_all = [stats_comm[s, :, H:2 * H] for s in range(N_DEV)]
        M = jnp.maximum(jnp.maximum(m_all[0], m_all[1]),
                        jnp.maximum(m_all[2], m_all[3]))
        L = (jnp.exp(m_all[0] - M) * l_all[0] + jnp.exp(m_all[1] - M) * l_all[1]
             + jnp.exp(m_all[2] - M) * l_all[2] + jnp.exp(m_all[3] - M) * l_all[3])
        alpha = jnp.exp(m_all[0] - M) / L
        for h in range(H):
            cs = slice(h * DH, (h + 1) * DH)
            o_buf[:, cs] = o_buf[:, cs] * alpha[:, h:h + 1]

    def rows(idx):
        return pl.ds(idx * CHUNK, CHUNK)

    def compute_chunk(idx, tag=""):
        with _scope(f"wo{tag}"):
            val = jnp.dot(o_buf[rows(idx), :], wo_ref[:, :],
                          preferred_element_type=jnp.float32)
            out_ref[0, rows(idx), :] = val
            obf[rows(idx), :] = val.astype(jnp.bfloat16)

    def start_rs(t):
        cw = pltpu.make_async_remote_copy(
            src_ref=obf.at[rows(jnp.mod(my - t, N_DEV)), pl.ds(0, HALF)],
            dst_ref=rs_cw.at[t],
            send_sem=send_cw, recv_sem=rs_recv_cw.at[t],
            device_id=(right,), device_id_type=MESH,
        )
        ccw = pltpu.make_async_remote_copy(
            src_ref=obf.at[rows(jnp.mod(my + t, N_DEV)), pl.ds(HALF, HALF)],
            dst_ref=rs_ccw.at[t],
            send_sem=send_ccw, recv_sem=rs_recv_ccw.at[t],
            device_id=(left,), device_id_type=MESH,
        )
        cw.start()
        ccw.start()
        return cw, ccw

    def finish_rs(t, cw, ccw):
        with _scope(f"rs_wait#t={t}"):
            cw.wait()
            ccw.wait()
        with _scope(f"rs_add#t={t}"):
            acw = jnp.mod(my - t - 1, N_DEV)
            vcw = (out_ref[0, rows(acw), pl.ds(0, HALF)]
                   + rs_cw[t, :, :].astype(jnp.float32))
            out_ref[0, rows(acw), pl.ds(0, HALF)] = vcw
            obf[rows(acw), pl.ds(0, HALF)] = vcw.astype(jnp.bfloat16)
            accw = jnp.mod(my + t + 1, N_DEV)
            vccw = (out_ref[0, rows(accw), pl.ds(HALF, HALF)]
                    + rs_ccw[t, :, :].astype(jnp.float32))
            out_ref[0, rows(accw), pl.ds(HALF, HALF)] = vccw
            obf[rows(accw), pl.ds(HALF, HALF)] = vccw.astype(jnp.bfloat16)

    compute_chunk(my, "#c=0")
    h0 = start_rs(0)
    compute_chunk(jnp.mod(my - 1, N_DEV), "#c=1")
    compute_chunk(jnp.mod(my + 1, N_DEV), "#c=2")
    finish_rs(0, *h0)
    h1 = start_rs(1)
    compute_chunk(diag, "#c=3")
    finish_rs(1, *h1)
    h2 = start_rs(2)
    finish_rs(2, *h2)

    for t in range(N_DEV - 1):
      with _scope(f"ag#t={t}"):
        fcw = jnp.mod(my + 1 - t, N_DEV)
        cw = pltpu.make_async_remote_copy(
            src_ref=obf.at[rows(fcw), pl.ds(0, HALF)],
            dst_ref=ag_cw.at[t],
            send_sem=send_cw, recv_sem=ag_recv_cw.at[t],
            device_id=(right,), device_id_type=MESH,
        )
        fccw = jnp.mod(my - 1 + t, N_DEV)
        ccw = pltpu.make_async_remote_copy(
            src_ref=obf.at[rows(fccw), pl.ds(HALF, HALF)],
            dst_ref=ag_ccw.at[t],
            send_sem=send_ccw, recv_sem=ag_recv_ccw.at[t],
            device_id=(left,), device_id_type=MESH,
        )
        cw.start()
        ccw.start()
        cw.wait()
        ccw.wait()
        rcw = jnp.mod(my - t, N_DEV)
        out_ref[0, rows(rcw), pl.ds(0, HALF)] = (
            ag_cw[t, :, :].astype(jnp.float32))
        obf[rows(rcw), pl.ds(0, HALF)] = ag_cw[t, :, :]
        rccw = jnp.mod(my + t, N_DEV)
        out_ref[0, rows(rccw), pl.ds(HALF, HALF)] = (
            ag_ccw[t, :, :].astype(jnp.float32))
        obf[rows(rccw), pl.ds(HALF, HALF)] = ag_ccw[t, :, :]


def kernel(x, Wq, K_ext, V_ext, Wo):
    return pl.pallas_call(
        _body,
        out_shape=jax.ShapeDtypeStruct((1, SQ, DM), jnp.float32),
        in_specs=[
            pl.BlockSpec(memory_space=pl.ANY),
            pl.BlockSpec(memory_space=pltpu.VMEM),
            pl.BlockSpec(memory_space=pl.ANY),
            pl.BlockSpec(memory_space=pl.ANY),
            pl.BlockSpec(memory_space=pltpu.VMEM),
        ],
        out_specs=pl.BlockSpec(memory_space=pltpu.VMEM),
        scratch_shapes=[
            pltpu.VMEM((2, CHUNK, DM), jnp.float32),
            pltpu.VMEM((CHUNK, DM), jnp.float32),
            pltpu.VMEM((2, CHUNK, H, DH), jnp.float32),
            pltpu.VMEM((2, CHUNK, H, DH), jnp.float32),
            pltpu.VMEM((SQ, DM), jnp.float32),
            pltpu.VMEM((CHUNK, 2 * H), jnp.float32),
            pltpu.VMEM((N_DEV, SQ, 2 * H), jnp.float32),
            pltpu.VMEM((SQ, DM), jnp.bfloat16),
            pltpu.SemaphoreType.DMA((2, 3)),
            pltpu.SemaphoreType.DMA((N_DEV - 1,)),
            pltpu.SemaphoreType.DMA((N_DEV - 1,)),
            pltpu.SemaphoreType.DMA,
            pltpu.SemaphoreType.DMA,
            pltpu.VMEM((N_DEV - 1, CHUNK, HALF), jnp.bfloat16),
            pltpu.VMEM((N_DEV - 1, CHUNK, HALF), jnp.bfloat16),
            pltpu.VMEM((N_DEV - 1, CHUNK, HALF), jnp.bfloat16),
            pltpu.VMEM((N_DEV - 1, CHUNK, HALF), jnp.bfloat16),
            pltpu.SemaphoreType.DMA((N_DEV - 1,)),
            pltpu.SemaphoreType.DMA((N_DEV - 1,)),
            pltpu.SemaphoreType.DMA((N_DEV - 1,)),
            pltpu.SemaphoreType.DMA((N_DEV - 1,)),
        ],
        compiler_params=pltpu.CompilerParams(
            collective_id=0, vmem_limit_bytes=59 * 1024 * 1024),
    )(x, Wq, K_ext, V_ext, Wo)


# device time: 119807 ns/iter; 1.0136x vs baseline; 1.0136x over previous
import contextlib

import jax
import jax.numpy as jnp
from jax import lax
from jax.experimental import pallas as pl
from jax.experimental.pallas import tpu as pltpu

_PROFILE_SCOPES = False


def _scope(name):
    if _PROFILE_SCOPES:
        return jax.named_scope(name)
    return contextlib.nullcontext()

N_DEV = 4
SQ = 2048
DM = 1024
H = 8
DH = 128
BLK = 64
R = 4
JB = 8
CHUNK = SQ // N_DEV
HALF = DM // 2
SCALE = 0.08838834764831843
MESH = pl.DeviceIdType.MESH


def _body(x_hbm, wq_ref, k_hbm, v_hbm, wo_ref, out_ref,
          xr, qr, kr, vr, o_buf, stats_r, stats_comm, obf,
          dma_sems, st_send, st_recv, send_cw, send_ccw,
          rs_cw, rs_ccw, ag_cw, ag_ccw,
          rs_recv_cw, rs_recv_ccw, ag_recv_cw, ag_recv_ccw):
    my = lax.axis_index("i")
    right = jnp.mod(my + 1, N_DEV)
    left = jnp.mod(my - 1, N_DEV)
    diag = jnp.mod(my + 2, N_DEV)

    def issue_copies(r):
        slot = r % 2
        copies = []
        for j in range(JB):
            b = r + R * j
            copies.append(pltpu.make_async_copy(
                x_hbm.at[0, pl.ds(b * BLK, BLK), :],
                xr.at[slot, pl.ds(j * BLK, BLK), :],
                dma_sems.at[slot, 0]))
            copies.append(pltpu.make_async_copy(
                k_hbm.at[0, pl.ds(b * BLK, BLK), :, :],
                kr.at[slot, pl.ds(j * BLK, BLK), :, :],
                dma_sems.at[slot, 1]))
            copies.append(pltpu.make_async_copy(
                v_hbm.at[0, pl.ds(b * BLK, BLK), :, :],
                vr.at[slot, pl.ds(j * BLK, BLK), :, :],
                dma_sems.at[slot, 2]))
        for c in copies:
            c.start()
        return copies

    pending = issue_copies(0)
    for r in range(R):
      with _scope(f"phA#r={r}"):
        slot = r % 2
        blocks = [r + R * j for j in range(JB)]
        nxt = issue_copies(r + 1) if r + 1 < R else []
        for c in pending:
            c.wait()
        pending = nxt

        qr[:, :] = jnp.dot(xr[slot], wq_ref[:, :],
                           preferred_element_type=jnp.float32) * SCALE
        for h in range(H):
            cs = slice(h * DH, (h + 1) * DH)
            s = lax.dot_general(qr[:, cs], kr[slot, :, h, :],
                                (((1,), (1,)), ((), ())),
                                preferred_element_type=jnp.float32)
            m = jnp.max(s, axis=1, keepdims=True)
            w = jnp.exp(s - m)
            l = jnp.sum(w, axis=1, keepdims=True)
            o = jnp.dot(w, vr[slot, :, h, :],
                        preferred_element_type=jnp.float32)
            for j, b in enumerate(blocks):
                o_buf[pl.ds(b * BLK, BLK), cs] = o[j * BLK:(j + 1) * BLK, :]
            stats_r[:, h:h + 1] = m
            stats_r[:, H + h:H + h + 1] = l
        for j, b in enumerate(blocks):
            stats_comm[0, pl.ds(b * BLK, BLK), :] = stats_r[pl.ds(j * BLK, BLK), :]

    with _scope("barrier"):
        barrier = pltpu.get_barrier_semaphore()
        for nbr in (left, right, diag):
            pl.semaphore_signal(barrier, inc=1, device_id=(nbr,),
                                device_id_type=MESH)
        pl.semaphore_wait(barrier, 3)

    with _scope("stats_x"):
        stat_sends = []
        for d in range(1, N_DEV):
            slot = N_DEV - d
            rdma = pltpu.make_async_remote_copy(
                src_ref=stats_comm.at[0],
                dst_ref=stats_comm.at[slot],
                send_sem=st_send.at[d - 1],
                recv_sem=st_recv.at[slot - 1],
                device_id=(jnp.mod(my + d, N_DEV),),
                device_id_type=MESH,
            )
            rdma.start()
            stat_sends.append(rdma)
        for k in range(1, N_DEV):
            pltpu.make_async_remote_copy(
                src_ref=stats_comm.at[k], dst_ref=stats_comm.at[k],
                send_sem=st_send.at[k - 1], recv_sem=st_recv.at[k - 1],
                device_id=(left,), device_id_type=MESH,
            ).wait_recv()
        for rdma in stat_sends:
            rdma.wait_send()

    with _scope("combine"):
        m_all = [stats_comm[s, :, 0:H] for s in range(N_DEV)]
        l_all = [stats_comm[s, :, H:2 * H] for s in range(N_DEV)]
        M = jnp.maximum(jnp.maximum(m_all[0], m_all[1]),
                        jnp.maximum(m_all[2], m_all[3]))
        L = (jnp.exp(m_all[0] - M) * l_all[0] + jnp.exp(m_all[1] - M) * l_all[1]
             + jnp.exp(m_all[2] - M) * l_all[2] + jnp.exp(m_all[3] - M) * l_all[3])
        alpha = jnp.exp(m_all[0] - M) / L
        for h in range(H):
            cs = slice(h * DH, (h + 1) * DH)
            o_buf[:, cs] = o_buf[:, cs] * alpha[:, h:h + 1]

    def rows(idx):
        return pl.ds(idx * CHUNK, CHUNK)

    def compute_chunk(idx, tag=""):
        with _scope(f"wo{tag}"):
            val = jnp.dot(o_buf[rows(idx), :], wo_ref[:, :],
                          preferred_element_type=jnp.float32)
            out_ref[0, rows(idx), :] = val
            obf[rows(idx), :] = val.astype(jnp.bfloat16)

    def start_rs(t):
        cw = pltpu.make_async_remote_copy(
            src_ref=obf.at[rows(jnp.mod(my - t, N_DEV)), pl.ds(0, HALF)],
            dst_ref=rs_cw.at[t],
            send_sem=send_cw, recv_sem=rs_recv_cw.at[t],
            device_id=(right,), device_id_type=MESH,
        )
        ccw = pltpu.make_async_remote_copy(
            src_ref=obf.at[rows(jnp.mod(my + t, N_DEV)), pl.ds(HALF, HALF)],
            dst_ref=rs_ccw.at[t],
            send_sem=send_ccw, recv_sem=rs_recv_ccw.at[t],
            device_id=(left,), device_id_type=MESH,
        )
        cw.start()
        ccw.start()
        return cw, ccw

    def finish_rs(t, cw, ccw):
        with _scope(f"rs_wait#t={t}"):
            cw.wait()
            ccw.wait()
        with _scope(f"rs_add#t={t}"):
            acw = jnp.mod(my - t - 1, N_DEV)
            vcw = (out_ref[0, rows(acw), pl.ds(0, HALF)]
                   + rs_cw[t, :, :].astype(jnp.float32))
            out_ref[0, rows(acw), pl.ds(0, HALF)] = vcw
            obf[rows(acw), pl.ds(0, HALF)] = vcw.astype(jnp.bfloat16)
            accw = jnp.mod(my + t + 1, N_DEV)
            vccw = (out_ref[0, rows(accw), pl.ds(HALF, HALF)]
                    + rs_ccw[t, :, :].astype(jnp.float32))
            out_ref[0, rows(accw), pl.ds(HALF, HALF)] = vccw
            obf[rows(accw), pl.ds(HALF, HALF)] = vccw.astype(jnp.bfloat16)

    compute_chunk(my, "#c=0")
    h0 = start_rs(0)
    compute_chunk(jnp.mod(my - 1, N_DEV), "#c=1")
    compute_chunk(jnp.mod(my + 1, N_DEV), "#c=2")
    finish_rs(0, *h0)
    h1 = start_rs(1)
    compute_chunk(diag, "#c=3")
    finish_rs(1, *h1)
    h2 = start_rs(2)
    finish_rs(2, *h2)

    for t in range(N_DEV - 1):
      with _scope(f"ag#t={t}"):
        fcw = jnp.mod(my + 1 - t, N_DEV)
        cw = pltpu.make_async_remote_copy(
            src_ref=obf.at[rows(fcw), pl.ds(0, HALF)],
            dst_ref=ag_cw.at[t],
            send_sem=send_cw, recv_sem=ag_recv_cw.at[t],
            device_id=(right,), device_id_type=MESH,
        )
        fccw = jnp.mod(my - 1 + t, N_DEV)
        ccw = pltpu.make_async_remote_copy(
            src_ref=obf.at[rows(fccw), pl.ds(HALF, HALF)],
            dst_ref=ag_ccw.at[t],
            send_sem=send_ccw, recv_sem=ag_recv_ccw.at[t],
            device_id=(left,), device_id_type=MESH,
        )
        cw.start()
        ccw.start()
        cw.wait()
        ccw.wait()
        rcw = jnp.mod(my - t, N_DEV)
        out_ref[0, rows(rcw), pl.ds(0, HALF)] = (
            ag_cw[t, :, :].astype(jnp.float32))
        obf[rows(rcw), pl.ds(0, HALF)] = ag_cw[t, :, :]
        rccw = jnp.mod(my + t, N_DEV)
        out_ref[0, rows(rccw), pl.ds(HALF, HALF)] = (
            ag_ccw[t, :, :].astype(jnp.float32))
        obf[rows(rccw), pl.ds(HALF, HALF)] = ag_ccw[t, :, :]


def kernel(x, Wq, K_ext, V_ext, Wo):
    return pl.pallas_call(
        _body,
        out_shape=jax.ShapeDtypeStruct((1, SQ, DM), jnp.float32),
        in_specs=[
            pl.BlockSpec(memory_space=pl.ANY),
            pl.BlockSpec(memory_space=pltpu.VMEM),
            pl.BlockSpec(memory_space=pl.ANY),
            pl.BlockSpec(memory_space=pl.ANY),
            pl.BlockSpec(memory_space=pltpu.VMEM),
        ],
        out_specs=pl.BlockSpec(memory_space=pltpu.VMEM),
        scratch_shapes=[
            pltpu.VMEM((2, CHUNK, DM), jnp.float32),
            pltpu.VMEM((CHUNK, DM), jnp.float32),
            pltpu.VMEM((2, CHUNK, H, DH), jnp.float32),
            pltpu.VMEM((2, CHUNK, H, DH), jnp.float32),
            pltpu.VMEM((SQ, DM), jnp.float32),
            pltpu.VMEM((CHUNK, 2 * H), jnp.float32),
            pltpu.VMEM((N_DEV, SQ, 2 * H), jnp.float32),
            pltpu.VMEM((SQ, DM), jnp.bfloat16),
            pltpu.SemaphoreType.DMA((2, 3)),
            pltpu.SemaphoreType.DMA((N_DEV - 1,)),
            pltpu.SemaphoreType.DMA((N_DEV - 1,)),
            pltpu.SemaphoreType.DMA,
            pltpu.SemaphoreType.DMA,
            pltpu.VMEM((N_DEV - 1, CHUNK, HALF), jnp.bfloat16),
            pltpu.VMEM((N_DEV - 1, CHUNK, HALF), jnp.bfloat16),
            pltpu.VMEM((N_DEV - 1, CHUNK, HALF), jnp.bfloat16),
            pltpu.VMEM((N_DEV - 1, CHUNK, HALF), jnp.bfloat16),
            pltpu.SemaphoreType.DMA((N_DEV - 1,)),
            pltpu.SemaphoreType.DMA((N_DEV - 1,)),
            pltpu.SemaphoreType.DMA((N_DEV - 1,)),
            pltpu.SemaphoreType.DMA((N_DEV - 1,)),
        ],
        compiler_params=pltpu.CompilerParams(
            collective_id=0, vmem_limit_bytes=59 * 1024 * 1024),
    )(x, Wq, K_ext, V_ext, Wo)
